# baseline (speedup 1.0000x reference)
import functools

import jax
import jax.numpy as jnp
from jax import lax
from jax.experimental import pallas as pl
from jax.experimental.pallas import tpu as pltpu
from jax.experimental.pallas import tpu_sc as plsc

EMBED = 32
B = 16384
H1 = 128
H2 = 64
NC = 2
NS = 16
NW = NC * NS
BPW = B // NW
CH = 128
NCH = BPW // CH
TSZ = 100000 * EMBED


def _sc_gather(uid, iid, tall, ub, ib):
    mesh = plsc.VectorSubcoreMesh(core_axis_name="c", subcore_axis_name="s")

    @functools.partial(
        pl.kernel,
        mesh=mesh,
        compiler_params=pltpu.CompilerParams(use_tc_tiling_on_sc=False),
        out_type=[
            jax.ShapeDtypeStruct((B * EMBED,), jnp.float32),
            jax.ShapeDtypeStruct((B * EMBED,), jnp.float32),
            jax.ShapeDtypeStruct((B * EMBED,), jnp.float32),
            jax.ShapeDtypeStruct((B * EMBED,), jnp.float32),
            jax.ShapeDtypeStruct((B,), jnp.float32),
            jax.ShapeDtypeStruct((B,), jnp.float32),
        ],
        scratch_types=[
            pltpu.VMEM((BPW,), jnp.int32),
            pltpu.VMEM((BPW,), jnp.int32),
            pltpu.VMEM((BPW + 16,), jnp.int32),
            pltpu.VMEM((BPW + 16,), jnp.int32),
            pltpu.VMEM((2, CH * EMBED), jnp.float32),
            pltpu.VMEM((2, CH * EMBED), jnp.float32),
            pltpu.VMEM((2, CH * EMBED), jnp.float32),
            pltpu.VMEM((2, CH * EMBED), jnp.float32),
            pltpu.VMEM((BPW,), jnp.float32),
            pltpu.VMEM((BPW,), jnp.float32),
            pltpu.SemaphoreType.DMA,
            pltpu.SemaphoreType.DMA,
            pltpu.SemaphoreType.DMA,
        ],
    )
    def k(uid_h, iid_h, tall_h, ub_h, ib_h,
          oug, oig, oum, oim, obu, obi,
          uidx, iidx, uofs, iofs,
          cug, cig, cum, cim, vbu, vbi, gsem, wsem, bsem):
        wid = lax.axis_index("s") * NC + lax.axis_index("c")
        base = wid * BPW
        pltpu.sync_copy(uid_h.at[pl.ds(base, BPW)], uidx)
        pltpu.sync_copy(iid_h.at[pl.ds(base, BPW)], iidx)
        for t in range(BPW // 16):
            sl = pl.ds(t * 16, 16)
            uofs[sl] = lax.shift_left(uidx[sl], 5)
            iofs[sl] = lax.shift_left(iidx[sl], 5)
        bias = []
        for j in range(NCH):
            sl = pl.ds(j * CH, CH)
            bias.append(pltpu.async_copy(ub_h.at[uidx.at[sl]], vbu.at[sl], bsem))
            bias.append(pltpu.async_copy(ib_h.at[iidx.at[sl]], vbi.at[sl], bsem))
        pend = {0: [], 1: []}
        for j in range(NCH):
            par = j % 2

            def body(r, carry, j=j, par=par):
                row = j * CH + r
                uo = pl.multiple_of(uofs[pl.ds(row, 16)][0], EMBED)
                io = pl.multiple_of(iofs[pl.ds(row, 16)][0], EMBED)
                d = pl.ds(r * EMBED, EMBED)
                pltpu.async_copy(tall_h.at[pl.ds(uo, EMBED)], cug.at[par, d], gsem)
                pltpu.async_copy(tall_h.at[pl.ds(uo + 2 * TSZ, EMBED)], cum.at[par, d], gsem)
                pltpu.async_copy(tall_h.at[pl.ds(io + TSZ, EMBED)], cig.at[par, d], gsem)
                pltpu.async_copy(tall_h.at[pl.ds(io + 3 * TSZ, EMBED)], cim.at[par, d], gsem)
                return carry

            lax.fori_loop(0, CH, body, 0)
            for _ in range(4):
                pltpu.make_async_copy(oug.at[pl.ds(0, CH * EMBED)],
                                      cug.at[par], gsem).wait()

            for w in pend[par]:
                w.wait()
            osl = pl.ds((base + j * CH) * EMBED, CH * EMBED)
            pend[par] = [
                pltpu.async_copy(cug.at[par], oug.at[osl], wsem),
                pltpu.async_copy(cig.at[par], oig.at[osl], wsem),
                pltpu.async_copy(cum.at[par], oum.at[osl], wsem),
                pltpu.async_copy(cim.at[par], oim.at[osl], wsem),
            ]
        for b in bias:
            b.wait()
        bw = [
            pltpu.async_copy(vbu, obu.at[pl.ds(base, BPW)], wsem),
            pltpu.async_copy(vbi, obi.at[pl.ds(base, BPW)], wsem),
        ]
        for par in (0, 1):
            for w in pend[par]:
                w.wait()
        for w in bw:
            w.wait()

    return k(uid, iid, tall, ub, ib)


def _tc_mlp(ug, ig, um, im, bu2, bi2, w1u, w1i, b1r, W2, b2r, wog, woh, bo):
    BLK = BPW
    G = B // BLK

    def body(ug_r, ig_r, um_r, im_r, bu_r, bi_r, w1u_r, w1i_r, b1_r,
             w2_r, b2_r, wog_r, woh_r, bo_r, out_r):
        g = ug_r[...] * ig_r[...]
        x1 = jnp.dot(um_r[...], w1u_r[...], preferred_element_type=jnp.float32)
        x1 = x1 + jnp.dot(im_r[...], w1i_r[...], preferred_element_type=jnp.float32)
        h1 = jnp.maximum(x1 + b1_r[...], 0.0)
        x2 = jnp.dot(h1, w2_r[...], preferred_element_type=jnp.float32)
        h2 = jnp.maximum(x2 + b2_r[...], 0.0)
        p = jnp.sum(g * wog_r[...], axis=1) + jnp.sum(h2 * woh_r[...], axis=1)
        out_r[...] = (p + bo_r[0]).reshape(1, 1, BLK) + bu_r[...] + bi_r[...]

    out = pl.pallas_call(
        body,
        grid=(G,),
        in_specs=[
            pl.BlockSpec((BLK, EMBED), lambda i: (i, 0)),
            pl.BlockSpec((BLK, EMBED), lambda i: (i, 0)),
            pl.BlockSpec((BLK, EMBED), lambda i: (i, 0)),
            pl.BlockSpec((BLK, EMBED), lambda i: (i, 0)),
            pl.BlockSpec((1, 1, BLK), lambda i: (i, 0, 0)),
            pl.BlockSpec((1, 1, BLK), lambda i: (i, 0, 0)),
            pl.BlockSpec((EMBED, H1), lambda i: (0, 0)),
            pl.BlockSpec((EMBED, H1), lambda i: (0, 0)),
            pl.BlockSpec((1, H1), lambda i: (0, 0)),
            pl.BlockSpec((H1, H2), lambda i: (0, 0)),
            pl.BlockSpec((1, H2), lambda i: (0, 0)),
            pl.BlockSpec((1, EMBED), lambda i: (0, 0)),
            pl.BlockSpec((1, H2), lambda i: (0, 0)),
            pl.BlockSpec(memory_space=pltpu.SMEM),
        ],
        out_specs=pl.BlockSpec((1, 1, BLK), lambda i: (i, 0, 0)),
        out_shape=jax.ShapeDtypeStruct((G, 1, BLK), jnp.float32),
    )(ug, ig, um, im, bu2, bi2, w1u, w1i, b1r, W2, b2r, wog, woh, bo)
    return out.reshape(B)


def kernel(user_ids, item_ids, Ug, Ig, Um, Im, Ub, Ib, W1, b1, W2, b2, Wo, bo):
    uid = user_ids.astype(jnp.int32)
    iid = item_ids.astype(jnp.int32)
    tall = jnp.stack([Ug, Ig, Um, Im], axis=0).reshape(-1)
    ug, ig, um, im, bu, bi = _sc_gather(
        uid, iid, tall, Ub.reshape(-1), Ib.reshape(-1))
    return _tc_mlp(
        ug.reshape(B, EMBED), ig.reshape(B, EMBED),
        um.reshape(B, EMBED), im.reshape(B, EMBED),
        bu.reshape(B // BPW, 1, BPW), bi.reshape(B // BPW, 1, BPW),
        W1[:EMBED], W1[EMBED:], b1.reshape(1, H1),
        W2, b2.reshape(1, H2),
        Wo[:EMBED].reshape(1, EMBED), Wo[EMBED:].reshape(1, H2), bo)
